# baseline (speedup 1.0000x reference)
import functools

import jax
import jax.numpy as jnp
from jax import lax
from jax.experimental import pallas as pl
from jax.experimental.pallas import tpu as pltpu
from jax.experimental.pallas import tpu_sc as plsc

_N_ROWS = 32768
_N_COLS = 2048
_CH = 8
_NW = 32
_LANE = 16
_STRIDE = _N_COLS + 8
_NBUF = 4
_UNROLL = 16
_NACC = 4


def _sc_rowmask_body(
    x_hbm, o_hbm,
    in0, in1, in2, in3, outa, outb, preva, prevb,
    s_in0, s_in1, s_in2, s_in3, s_oa, s_ob,
):
    wid = lax.axis_index("s") * 2 + lax.axis_index("c")
    rpw = _N_ROWS // _NW
    nch = rpw // _CH
    base_row = wid * rpw
    chunk_words = _CH * _N_COLS

    lane = lax.iota(jnp.int32, _LANE)
    zero_f = jnp.zeros((_LANE,), jnp.float32)
    row8 = lane & 7
    gbase = row8 * _STRIDE
    sbase = row8 * _N_COLS

    ins = (in0, in1, in2, in3)
    sins = (s_in0, s_in1, s_in2, s_in3)
    outs = (outa, outb)
    prevs = (preva, prevb)
    souts = (s_oa, s_ob)

    def _zero_blk(i, _):
        outa[pl.ds(i * _LANE, _LANE)] = zero_f
        outb[pl.ds(i * _LANE, _LANE)] = zero_f
        return 0

    lax.fori_loop(0, chunk_words // _LANE, _zero_blk, 0)
    preva[...] = sbase
    prevb[...] = sbase

    def _start_in(c, b):
        w0 = (base_row + c * _CH) * _N_COLS
        for r in range(_CH):
            pltpu.make_async_copy(
                x_hbm.at[pl.ds(w0 + r * _N_COLS, _N_COLS)],
                ins[b].at[pl.ds(r * _STRIDE, _N_COLS)],
                sins[b],
            ).start()

    def _wait_in(b):
        for r in range(_CH):
            pltpu.make_async_copy(
                x_hbm.at[pl.ds(base_row * _N_COLS + r * _N_COLS, _N_COLS)],
                ins[b].at[pl.ds(r * _STRIDE, _N_COLS)],
                sins[b],
            ).wait()

    def _compute_chunk(b):
        inb = ins[b]

        def _cols(i, carry):
            ms, mis = carry
            ms, mis = list(ms), list(mis)
            tbase = i * _UNROLL + lane
            for k in range(_UNROLL):
                a = k % _NACC
                jcol = (tbase + k) & (_N_COLS - 1)
                v = plsc.load_gather(inb, [gbase + jcol])
                upd = (v > ms[a]) | ((v == ms[a]) & (jcol < mis[a]))
                ms[a] = jnp.where(upd, v, ms[a])
                mis[a] = jnp.where(upd, jcol, mis[a])
            return (tuple(ms), tuple(mis))

        m0 = tuple(jnp.full((_LANE,), -jnp.inf, jnp.float32) for _ in range(_NACC))
        mi0 = tuple(jnp.zeros((_LANE,), jnp.int32) for _ in range(_NACC))
        ms, mis = lax.fori_loop(0, _N_COLS // _UNROLL, _cols, (m0, mi0))
        m, mi = ms[0], mis[0]
        for a in range(1, _NACC):
            upd = (ms[a] > m) | ((ms[a] == m) & (mis[a] < mi))
            m = jnp.where(upd, ms[a], m)
            mi = jnp.where(upd, mis[a], mi)
        return (m, mi)

    def _do_chunk(c, h, b):
        @pl.when(c + _NBUF - 1 < nch)
        def _():
            _start_in(c + _NBUF - 1, (b + _NBUF - 1) % _NBUF)

        _wait_in(b)
        valv, colv = _compute_chunk(b)
        ob, pb, sb = outs[h], prevs[h], souts[h]

        @pl.when(c >= 2)
        def _():
            pltpu.make_async_copy(
                ob, o_hbm.at[pl.ds(base_row * _N_COLS, chunk_words)], sb
            ).wait()

        pv = pb[...]
        plsc.store_scatter(ob, [pv], zero_f)
        idxv = sbase + colv
        plsc.store_scatter(ob, [idxv], valv)
        pb[...] = idxv
        w0 = (base_row + c * _CH) * _N_COLS
        pltpu.make_async_copy(ob, o_hbm.at[pl.ds(w0, chunk_words)], sb).start()

    for b in range(_NBUF - 1):
        _start_in(b, b)

    def _grp(p, _):
        c0 = p * _NBUF
        for b in range(_NBUF):
            _do_chunk(c0 + b, b % 2, b)
        return 0

    lax.fori_loop(0, nch // _NBUF, _grp, 0)
    for h in range(2):
        pltpu.make_async_copy(
            outs[h], o_hbm.at[pl.ds(base_row * _N_COLS, chunk_words)], souts[h]
        ).wait()


@functools.partial(jax.jit, static_argnames=())
def kernel(x):
    mesh = plsc.VectorSubcoreMesh(
        core_axis_name="c", subcore_axis_name="s", num_cores=2, num_subcores=16
    )
    sc = pl.kernel(
        _sc_rowmask_body,
        mesh=mesh,
        compiler_params=pltpu.CompilerParams(needs_layout_passes=False),
        out_type=jax.ShapeDtypeStruct((_N_ROWS * _N_COLS,), jnp.float32),
        scratch_types=[
            pltpu.VMEM((_CH * _STRIDE,), jnp.float32),
            pltpu.VMEM((_CH * _STRIDE,), jnp.float32),
            pltpu.VMEM((_CH * _STRIDE,), jnp.float32),
            pltpu.VMEM((_CH * _STRIDE,), jnp.float32),
            pltpu.VMEM((_CH * _N_COLS,), jnp.float32),
            pltpu.VMEM((_CH * _N_COLS,), jnp.float32),
            pltpu.VMEM((_LANE,), jnp.int32),
            pltpu.VMEM((_LANE,), jnp.int32),
            pltpu.SemaphoreType.DMA,
            pltpu.SemaphoreType.DMA,
            pltpu.SemaphoreType.DMA,
            pltpu.SemaphoreType.DMA,
            pltpu.SemaphoreType.DMA,
            pltpu.SemaphoreType.DMA,
        ],
    )
    return sc(x.reshape(-1)).reshape(_N_ROWS, _N_COLS)


# speedup vs baseline: 1.3283x; 1.3283x over previous
import functools

import jax
import jax.numpy as jnp
from jax import lax
from jax.experimental import pallas as pl
from jax.experimental.pallas import tpu as pltpu
from jax.experimental.pallas import tpu_sc as plsc

_N_ROWS = 32768
_N_COLS = 2048
_CH = 16
_NW = 32
_LANE = 16
_STRIDE = _N_COLS + 16
_UNROLL = 16
_NACC = 4


def _sc_rowmask_body(x_hbm, o_hbm, in0, in1, outb, previdx, s_in0, s_in1, s_out):
    wid = lax.axis_index("s") * 2 + lax.axis_index("c")
    rpw = _N_ROWS // _NW
    nch = rpw // _CH
    base_row = wid * rpw
    chunk_words = _CH * _N_COLS

    lane = lax.iota(jnp.int32, _LANE)
    zero_f = jnp.zeros((_LANE,), jnp.float32)
    sbase = lane * _N_COLS
    gbase = lane * _STRIDE

    def _zero_blk(i, _):
        outb[pl.ds(i * _LANE, _LANE)] = zero_f
        return 0

    lax.fori_loop(0, chunk_words // _LANE, _zero_blk, 0)
    previdx[...] = lane

    ins = (in0, in1)
    sins = (s_in0, s_in1)

    def _start_in(c, b):
        w0 = (base_row + c * _CH) * _N_COLS
        for r in range(_CH):
            pltpu.make_async_copy(
                x_hbm.at[pl.ds(w0 + r * _N_COLS, _N_COLS)],
                ins[b].at[pl.ds(r * _STRIDE, _N_COLS)],
                sins[b],
            ).start()

    def _wait_in(b):
        for r in range(_CH):
            pltpu.make_async_copy(
                x_hbm.at[pl.ds(base_row * _N_COLS + r * _N_COLS, _N_COLS)],
                ins[b].at[pl.ds(r * _STRIDE, _N_COLS)],
                sins[b],
            ).wait()

    def _compute_chunk(b):
        inb = ins[b]

        def _cols(i, carry):
            ms, mis = carry
            ms, mis = list(ms), list(mis)
            tbase = i * _UNROLL + lane
            for k in range(_UNROLL):
                a = k % _NACC
                jcol = (tbase + k) & (_N_COLS - 1)
                v = plsc.load_gather(inb, [gbase + jcol])
                upd = (v > ms[a]) | ((v == ms[a]) & (jcol < mis[a]))
                ms[a] = jnp.where(upd, v, ms[a])
                mis[a] = jnp.where(upd, jcol, mis[a])
            return (tuple(ms), tuple(mis))

        m0 = tuple(jnp.full((_LANE,), -jnp.inf, jnp.float32) for _ in range(_NACC))
        mi0 = tuple(jnp.zeros((_LANE,), jnp.int32) for _ in range(_NACC))
        ms, mis = lax.fori_loop(0, _N_COLS // _UNROLL, _cols, (m0, mi0))
        m, mi = ms[0], mis[0]
        for a in range(1, _NACC):
            upd = (ms[a] > m) | ((ms[a] == m) & (mis[a] < mi))
            m = jnp.where(upd, ms[a], m)
            mi = jnp.where(upd, mis[a], mi)
        return (m, mi)

    def _do_chunk(c, b):
        @pl.when(c + 1 < nch)
        def _():
            _start_in(c + 1, 1 - b)

        _wait_in(b)
        valv, colv = _compute_chunk(b)

        @pl.when(c > 0)
        def _():
            pltpu.make_async_copy(
                outb, o_hbm.at[pl.ds(base_row * _N_COLS, chunk_words)], s_out
            ).wait()

        pv = previdx[...]
        plsc.store_scatter(outb, [pv], zero_f)
        idxv = sbase + colv
        plsc.store_scatter(outb, [idxv], valv)
        previdx[...] = idxv
        w0 = (base_row + c * _CH) * _N_COLS
        pltpu.make_async_copy(outb, o_hbm.at[pl.ds(w0, chunk_words)], s_out).start()

    _start_in(0, 0)

    def _pair(p, _):
        c0 = p * 2
        _do_chunk(c0, 0)
        _do_chunk(c0 + 1, 1)
        return 0

    lax.fori_loop(0, nch // 2, _pair, 0)
    pltpu.make_async_copy(
        outb, o_hbm.at[pl.ds(base_row * _N_COLS, chunk_words)], s_out
    ).wait()


@functools.partial(jax.jit, static_argnames=())
def kernel(x):
    mesh = plsc.VectorSubcoreMesh(
        core_axis_name="c", subcore_axis_name="s", num_cores=2, num_subcores=16
    )
    sc = pl.kernel(
        _sc_rowmask_body,
        mesh=mesh,
        compiler_params=pltpu.CompilerParams(needs_layout_passes=False),
        out_type=jax.ShapeDtypeStruct((_N_ROWS * _N_COLS,), jnp.float32),
        scratch_types=[
            pltpu.VMEM((_CH * _STRIDE,), jnp.float32),
            pltpu.VMEM((_CH * _STRIDE,), jnp.float32),
            pltpu.VMEM((_CH * _N_COLS,), jnp.float32),
            pltpu.VMEM((_LANE,), jnp.int32),
            pltpu.SemaphoreType.DMA,
            pltpu.SemaphoreType.DMA,
            pltpu.SemaphoreType.DMA,
        ],
    )
    return sc(x.reshape(-1)).reshape(_N_ROWS, _N_COLS)


# speedup vs baseline: 1.3317x; 1.0026x over previous
import functools

import jax
import jax.numpy as jnp
from jax import lax
from jax.experimental import pallas as pl
from jax.experimental.pallas import tpu as pltpu
from jax.experimental.pallas import tpu_sc as plsc

_N_ROWS = 32768
_N_COLS = 2048
_CH = 16
_NW = 32
_LANE = 16
_STRIDE = _N_COLS + 16
_UNROLL = 16
_NACC = 4


def _sc_rowmask_body(x_hbm, o_hbm, in0, in1, zbuf, valbuf, s_in0, s_in1, s_z, s_w):
    wid = lax.axis_index("s") * 2 + lax.axis_index("c")
    rpw = _N_ROWS // _NW
    nch = rpw // _CH
    base_row = wid * rpw
    chunk_words = _CH * _N_COLS

    lane = lax.iota(jnp.int32, _LANE)
    zero_f = jnp.zeros((_LANE,), jnp.float32)
    sbase = lane * _N_COLS
    gbase = lane * _STRIDE

    def _zero_blk(i, _):
        zbuf[pl.ds(i * _LANE, _LANE)] = zero_f
        return 0

    lax.fori_loop(0, chunk_words // _LANE, _zero_blk, 0)

    ins = (in0, in1)
    sins = (s_in0, s_in1)

    def _start_in(c, b):
        w0 = (base_row + c * _CH) * _N_COLS
        for r in range(_CH):
            pltpu.make_async_copy(
                x_hbm.at[pl.ds(w0 + r * _N_COLS, _N_COLS)],
                ins[b].at[pl.ds(r * _STRIDE, _N_COLS)],
                sins[b],
            ).start()

    def _wait_in(b):
        for r in range(_CH):
            pltpu.make_async_copy(
                x_hbm.at[pl.ds(base_row * _N_COLS + r * _N_COLS, _N_COLS)],
                ins[b].at[pl.ds(r * _STRIDE, _N_COLS)],
                sins[b],
            ).wait()

    def _compute_chunk(b):
        inb = ins[b]

        def _cols(i, carry):
            ms, mis = carry
            ms, mis = list(ms), list(mis)
            tbase = i * _UNROLL + lane
            for k in range(_UNROLL):
                a = k % _NACC
                jcol = (tbase + k) & (_N_COLS - 1)
                v = plsc.load_gather(inb, [gbase + jcol])
                upd = (v > ms[a]) | ((v == ms[a]) & (jcol < mis[a]))
                ms[a] = jnp.where(upd, v, ms[a])
                mis[a] = jnp.where(upd, jcol, mis[a])
            return (tuple(ms), tuple(mis))

        m0 = tuple(jnp.full((_LANE,), -jnp.inf, jnp.float32) for _ in range(_NACC))
        mi0 = tuple(jnp.zeros((_LANE,), jnp.int32) for _ in range(_NACC))
        ms, mis = lax.fori_loop(0, _N_COLS // _UNROLL, _cols, (m0, mi0))
        m, mi = ms[0], mis[0]
        for a in range(1, _NACC):
            upd = (ms[a] > m) | ((ms[a] == m) & (mis[a] < mi))
            m = jnp.where(upd, ms[a], m)
            mi = jnp.where(upd, mis[a], mi)
        return (m, mi)

    def _do_chunk(c, b):
        w0 = (base_row + c * _CH) * _N_COLS
        pltpu.make_async_copy(zbuf, o_hbm.at[pl.ds(w0, chunk_words)], s_z).start()

        @pl.when(c + 1 < nch)
        def _():
            _start_in(c + 1, 1 - b)

        _wait_in(b)
        valv, colv = _compute_chunk(b)

        @pl.when(c > 0)
        def _():
            pltpu.make_async_copy(valbuf, o_hbm.at[lane], s_w).wait()

        valbuf[...] = valv
        idxv = (base_row + c * _CH + lane) * _N_COLS + colv
        pltpu.make_async_copy(zbuf, o_hbm.at[pl.ds(w0, chunk_words)], s_z).wait()
        pltpu.make_async_copy(valbuf, o_hbm.at[idxv], s_w).start()

    _start_in(0, 0)

    def _pair(p, _):
        c0 = p * 2
        _do_chunk(c0, 0)
        _do_chunk(c0 + 1, 1)
        return 0

    lax.fori_loop(0, nch // 2, _pair, 0)
    pltpu.make_async_copy(valbuf, o_hbm.at[lane], s_w).wait()


@functools.partial(jax.jit, static_argnames=())
def kernel(x):
    mesh = plsc.VectorSubcoreMesh(
        core_axis_name="c", subcore_axis_name="s", num_cores=2, num_subcores=16
    )
    sc = pl.kernel(
        _sc_rowmask_body,
        mesh=mesh,
        compiler_params=pltpu.CompilerParams(needs_layout_passes=False),
        out_type=jax.ShapeDtypeStruct((_N_ROWS * _N_COLS,), jnp.float32),
        scratch_types=[
            pltpu.VMEM((_CH * _STRIDE,), jnp.float32),
            pltpu.VMEM((_CH * _STRIDE,), jnp.float32),
            pltpu.VMEM((_CH * _N_COLS,), jnp.float32),
            pltpu.VMEM((_LANE,), jnp.float32),
            pltpu.SemaphoreType.DMA,
            pltpu.SemaphoreType.DMA,
            pltpu.SemaphoreType.DMA,
            pltpu.SemaphoreType.DMA,
        ],
    )
    return sc(x.reshape(-1)).reshape(_N_ROWS, _N_COLS)
